# baseline (speedup 1.0000x reference)
import functools

import jax
import jax.numpy as jnp
from jax import lax
from jax.experimental import pallas as pl
from jax.experimental.pallas import tpu as pltpu
from jax.experimental.pallas import tpu_sc as plsc

OUT_DIM = 64
NUM_NODE_TYPES = 5
FEAT = 128
B = 16384
NUM_CORES = 2
NUM_SUBCORES = 16
NW = NUM_CORES * NUM_SUBCORES
BPW = B // NW

_NBUF = 4
_Q = BPW // _NBUF
_NT = 3


def _sc_gather_body(batch_hbm, nt_hbm, fm_hbm, t0_hbm, t1_hbm, t2_hbm,
                    ty_out, g_out,
                    bidx_v, ty_v, fi_v, rows_v,
                    mi0, mi1, mi2, ps0, ps1, ps2,
                    sem_nt, sem_fm, sem_ty,
                    gs0, gs1, gs2, gs3, ws0, ws1, ws2, ws3):
  gsems = (gs0, gs1, gs2, gs3)
  wsems = (ws0, ws1, ws2, ws3)
  midx = (mi0, mi1, mi2)
  pos2 = (ps0, ps1, ps2)
  wid = lax.axis_index("s") * NUM_CORES + lax.axis_index("c")
  base = wid * BPW
  pltpu.sync_copy(batch_hbm.at[pl.ds(base, BPW)], bidx_v)
  h_nt = pltpu.async_copy(nt_hbm.at[bidx_v], ty_v, sem_nt)
  h_fm = pltpu.async_copy(fm_hbm.at[bidx_v], fi_v, sem_fm)
  h_nt.wait()
  h_ty = pltpu.async_copy(ty_v, ty_out.at[pl.ds(base, BPW)], sem_ty)
  h_fm.wait()

  lane = lax.iota(jnp.int32, 16)
  for j in range(BPW // 16):
    tyv = ty_v[pl.ds(j * 16, 16)]
    fiv = fi_v[pl.ds(j * 16, 16)]
    posb = base + j * 16 + lane
    for t in range(_NT):
      m = tyv == t
      midx[t][pl.ds(j * 16, 16)] = jnp.where(m, fiv, 0)
      pos2[t][j // 8, pl.ds((j % 8) * 16, 16)] = jnp.where(m, posb, posb + B)

  tables = (t0_hbm, t1_hbm, t2_hbm)
  nchunk = _NT * _NBUF

  def start_gather(k):
    t, q = k // _NBUF, k % _NBUF
    return pltpu.async_copy(
        tables[t].at[midx[t].at[pl.ds(q * _Q, _Q)]],
        rows_v.at[pl.ds((k % _NBUF) * _Q, _Q)], gsems[k % _NBUF])

  gh = [None] * nchunk
  wh = [None] * nchunk
  gh[0] = start_gather(0)
  for k in range(nchunk):
    if k + 1 < nchunk:
      if k + 1 >= _NBUF:
        wh[k + 1 - _NBUF].wait()
      gh[k + 1] = start_gather(k + 1)
    gh[k].wait()
    t, q = k // _NBUF, k % _NBUF
    wh[k] = pltpu.async_copy(
        rows_v.at[pl.ds((k % _NBUF) * _Q, _Q)],
        g_out.at[pos2[t].at[q]], wsems[k % _NBUF])
  for k in range(nchunk - _NBUF, nchunk):
    wh[k].wait()
  h_ty.wait()


@functools.cache
def _sc_gather():
  return pl.kernel(
      _sc_gather_body,
      out_type=[
          jax.ShapeDtypeStruct((B,), jnp.int32),
          jax.ShapeDtypeStruct((2 * B, FEAT), jnp.float32),
      ],
      mesh=plsc.VectorSubcoreMesh(core_axis_name="c", subcore_axis_name="s"),
      scratch_types=[
          pltpu.VMEM((BPW,), jnp.int32),
          pltpu.VMEM((BPW,), jnp.int32),
          pltpu.VMEM((BPW,), jnp.int32),
          pltpu.VMEM((BPW, FEAT), jnp.float32),
          pltpu.VMEM((BPW,), jnp.int32),
          pltpu.VMEM((BPW,), jnp.int32),
          pltpu.VMEM((BPW,), jnp.int32),
          pltpu.VMEM((_NBUF, _Q), jnp.int32),
          pltpu.VMEM((_NBUF, _Q), jnp.int32),
          pltpu.VMEM((_NBUF, _Q), jnp.int32),
      ] + [pltpu.SemaphoreType.DMA] * 11,
  )


_K2 = 2 * FEAT
_N2 = _NT * OUT_DIM
_BTC = 4096
_GRID = B // _BTC


def _tc_project_body(ty_ref, g_ref, w_ref, o_ref):
  ty = ty_ref[...]
  big = jnp.concatenate(
      [g_ref[...], jnp.ones((_BTC, FEAT), jnp.float32)], axis=1)
  p = jnp.dot(big, w_ref[...], preferred_element_type=jnp.float32)
  lanes = lax.broadcasted_iota(jnp.int32, (_BTC, OUT_DIM), 1)
  acc = (lanes == ty).astype(jnp.float32)
  for t in range(_NT):
    acc = acc + jnp.where(ty == t, p[:, t * OUT_DIM:(t + 1) * OUT_DIM], 0.0)
  o_ref[...] = acc


def _tc_project(ty_col, g, wc):
  return pl.pallas_call(
      _tc_project_body,
      grid=(_GRID,),
      in_specs=[
          pl.BlockSpec((_BTC, 1), lambda i: (i, 0)),
          pl.BlockSpec((_BTC, FEAT), lambda i: (i, 0)),
          pl.BlockSpec((_K2, _N2), lambda i: (0, 0)),
      ],
      out_specs=pl.BlockSpec((_BTC, OUT_DIM), lambda i: (i, 0)),
      out_shape=jax.ShapeDtypeStruct((B, OUT_DIM), jnp.float32),
  )(ty_col, g, wc)


def kernel(batch, node_types, feat_map, ip_feats, domain_feats, url_feats,
           W_ip, b_ip, W_dom, b_dom, W_url, b_url):
  batch = batch.astype(jnp.int32)
  node_types = node_types.astype(jnp.int32)
  feat_map = feat_map.astype(jnp.int32)

  ty_b, g = _sc_gather()(batch, node_types, feat_map,
                         ip_feats, domain_feats, url_feats)

  blocks = []
  for w, b in ((W_ip, b_ip), (W_dom, b_dom), (W_url, b_url)):
    top = jnp.pad(w.T.astype(jnp.float32), ((0, 0), (NUM_NODE_TYPES, 0)))
    brow = jnp.pad(b.astype(jnp.float32)[None, :],
                   ((0, 0), (NUM_NODE_TYPES, 0)))
    blocks.append(jnp.concatenate(
        [top, brow, jnp.zeros((FEAT - 1, OUT_DIM), jnp.float32)], axis=0))
  wc = jnp.concatenate(blocks, axis=1)

  ty_col = ty_b.reshape(B, 1)
  return _tc_project(ty_col, g, wc)


# speedup vs baseline: 20.4953x; 20.4953x over previous
import functools

import jax
import jax.numpy as jnp
from jax import lax
from jax.experimental import pallas as pl
from jax.experimental.pallas import tpu as pltpu
from jax.experimental.pallas import tpu_sc as plsc

OUT_DIM = 64
NUM_NODE_TYPES = 5
FEAT = 128
B = 16384
NUM_CORES = 2
NUM_SUBCORES = 16
NW = NUM_CORES * NUM_SUBCORES
BPW = B // NW

_NBUF = 4
_Q = BPW // _NBUF
_NT = 3


def _sc_gather_body(batch_hbm, nt_hbm, fm_hbm, t0_hbm, t1_hbm, t2_hbm,
                    ty_out, g_out,
                    bidx_v, ty_v, fi_v, rows_v,
                    mi0, mi1, mi2, ps0, ps1, ps2,
                    sem_nt, sem_fm, sem_ty,
                    gs0, gs1, gs2, gs3, ws0, ws1, ws2, ws3):
  gsems = (gs0, gs1, gs2, gs3)
  wsems = (ws0, ws1, ws2, ws3)
  midx = (mi0, mi1, mi2)
  pos2 = (ps0, ps1, ps2)
  wid = lax.axis_index("s") * NUM_CORES + lax.axis_index("c")
  base = wid * BPW
  pltpu.sync_copy(batch_hbm.at[pl.ds(base, BPW)], bidx_v)
  h_nt = pltpu.async_copy(nt_hbm.at[bidx_v], ty_v, sem_nt)
  h_fm = pltpu.async_copy(fm_hbm.at[bidx_v], fi_v, sem_fm)
  h_nt.wait()
  h_ty = pltpu.async_copy(ty_v, ty_out.at[pl.ds(base, BPW)], sem_ty)
  h_fm.wait()

  lane = lax.iota(jnp.int32, 16)
  for j in range(BPW // 16):
    tyv = ty_v[pl.ds(j * 16, 16)]
    fiv = fi_v[pl.ds(j * 16, 16)]
    posb = base + j * 16 + lane
    for t in range(_NT):
      m = tyv == t
      midx[t][pl.ds(j * 16, 16)] = fiv
      pos2[t][j // 8, pl.ds((j % 8) * 16, 16)] = jnp.where(m, posb, posb + B)

  tables = (t0_hbm, t1_hbm, t2_hbm)
  nchunk = _NT * _NBUF

  def start_gather(k):
    t, q = k // _NBUF, k % _NBUF
    return pltpu.async_copy(
        tables[t].at[midx[t].at[pl.ds(q * _Q, _Q)]],
        rows_v.at[pl.ds((k % _NBUF) * _Q, _Q)], gsems[k % _NBUF])

  gh = [None] * nchunk
  wh = [None] * nchunk
  depth = _NBUF - 1
  for k in range(depth):
    gh[k] = start_gather(k)
  for k in range(nchunk):
    if k + depth < nchunk:
      if k + depth >= _NBUF:
        wh[k + depth - _NBUF].wait()
      gh[k + depth] = start_gather(k + depth)
    gh[k].wait()
    t, q = k // _NBUF, k % _NBUF
    wh[k] = pltpu.async_copy(
        rows_v.at[pl.ds((k % _NBUF) * _Q, _Q)],
        g_out.at[pos2[t].at[q]], wsems[k % _NBUF])
  for k in range(nchunk - _NBUF, nchunk):
    wh[k].wait()
  h_ty.wait()


@functools.cache
def _sc_gather():
  return pl.kernel(
      _sc_gather_body,
      out_type=[
          jax.ShapeDtypeStruct((B,), jnp.int32),
          jax.ShapeDtypeStruct((2 * B, FEAT), jnp.float32),
      ],
      mesh=plsc.VectorSubcoreMesh(core_axis_name="c", subcore_axis_name="s"),
      scratch_types=[
          pltpu.VMEM((BPW,), jnp.int32),
          pltpu.VMEM((BPW,), jnp.int32),
          pltpu.VMEM((BPW,), jnp.int32),
          pltpu.VMEM((BPW, FEAT), jnp.float32),
          pltpu.VMEM((BPW,), jnp.int32),
          pltpu.VMEM((BPW,), jnp.int32),
          pltpu.VMEM((BPW,), jnp.int32),
          pltpu.VMEM((_NBUF, _Q), jnp.int32),
          pltpu.VMEM((_NBUF, _Q), jnp.int32),
          pltpu.VMEM((_NBUF, _Q), jnp.int32),
      ] + [pltpu.SemaphoreType.DMA] * 11,
  )


_K2 = 2 * FEAT
_N2 = _NT * OUT_DIM
_BTC = 4096
_GRID = B // _BTC


def _tc_project_body(ty_ref, g_ref, w_ref, o_ref):
  ty = ty_ref[...]
  big = jnp.concatenate(
      [g_ref[...], jnp.ones((_BTC, FEAT), jnp.float32)], axis=1)
  p = jnp.dot(big, w_ref[...], preferred_element_type=jnp.float32)
  lanes = lax.broadcasted_iota(jnp.int32, (_BTC, OUT_DIM), 1)
  acc = (lanes == ty).astype(jnp.float32)
  for t in range(_NT):
    acc = acc + jnp.where(ty == t, p[:, t * OUT_DIM:(t + 1) * OUT_DIM], 0.0)
  o_ref[...] = acc


def _tc_project(ty_col, g, wc):
  return pl.pallas_call(
      _tc_project_body,
      grid=(_GRID,),
      in_specs=[
          pl.BlockSpec((_BTC, 1), lambda i: (i, 0)),
          pl.BlockSpec((_BTC, FEAT), lambda i: (i, 0)),
          pl.BlockSpec((_K2, _N2), lambda i: (0, 0)),
      ],
      out_specs=pl.BlockSpec((_BTC, OUT_DIM), lambda i: (i, 0)),
      out_shape=jax.ShapeDtypeStruct((B, OUT_DIM), jnp.float32),
  )(ty_col, g, wc)


def kernel(batch, node_types, feat_map, ip_feats, domain_feats, url_feats,
           W_ip, b_ip, W_dom, b_dom, W_url, b_url):
  batch = batch.astype(jnp.int32)
  node_types = node_types.astype(jnp.int32)
  feat_map = feat_map.astype(jnp.int32)

  ty_b, g = _sc_gather()(batch, node_types, feat_map,
                         ip_feats, domain_feats, url_feats)

  blocks = []
  for w, b in ((W_ip, b_ip), (W_dom, b_dom), (W_url, b_url)):
    top = jnp.pad(w.T.astype(jnp.float32), ((0, 0), (NUM_NODE_TYPES, 0)))
    brow = jnp.pad(b.astype(jnp.float32)[None, :],
                   ((0, 0), (NUM_NODE_TYPES, 0)))
    blocks.append(jnp.concatenate(
        [top, brow, jnp.zeros((FEAT - 1, OUT_DIM), jnp.float32)], axis=0))
  wc = jnp.concatenate(blocks, axis=1)

  ty_col = ty_b.reshape(B, 1)
  return _tc_project(ty_col, g, wc)


# speedup vs baseline: 20.8353x; 1.0166x over previous
import functools

import jax
import jax.numpy as jnp
from jax import lax
from jax.experimental import pallas as pl
from jax.experimental.pallas import tpu as pltpu
from jax.experimental.pallas import tpu_sc as plsc

OUT_DIM = 64
NUM_NODE_TYPES = 5
FEAT = 128
B = 16384
NUM_CORES = 2
NUM_SUBCORES = 16
NW = NUM_CORES * NUM_SUBCORES
BPW = B // NW

_NBUF = 4
_Q = BPW // _NBUF
_NT = 3


def _sc_gather_body(batch_hbm, nt_hbm, fm_hbm, t0_hbm, t1_hbm, t2_hbm,
                    ty_out, g_out,
                    bidx_v, ty_v, fi_v, rows_v,
                    mi0, mi1, mi2, ps0, ps1, ps2,
                    sem_nt, sem_fm, sem_ty,
                    gs0, gs1, gs2, gs3, ws0, ws1, ws2, ws3):
  gsems = (gs0, gs1, gs2, gs3)
  wsems = (ws0, ws1, ws2, ws3)
  midx = (mi0, mi1, mi2)
  pos2 = (ps0, ps1, ps2)
  wid = lax.axis_index("s") * NUM_CORES + lax.axis_index("c")
  base = wid * BPW
  pltpu.sync_copy(batch_hbm.at[pl.ds(base, BPW)], bidx_v)
  h_nt = pltpu.async_copy(nt_hbm.at[bidx_v], ty_v, sem_nt)
  h_fm = pltpu.async_copy(fm_hbm.at[bidx_v], fi_v, sem_fm)
  h_nt.wait()
  h_ty = pltpu.async_copy(ty_v, ty_out.at[pl.ds(base, BPW)], sem_ty)
  h_fm.wait()

  lane = lax.iota(jnp.int32, 16)
  for j in range(BPW // 16):
    tyv = ty_v[pl.ds(j * 16, 16)]
    fiv = fi_v[pl.ds(j * 16, 16)]
    posb = base + j * 16 + lane
    for t in range(_NT):
      m = tyv == t
      midx[t][pl.ds(j * 16, 16)] = fiv
      pos2[t][j // 8, pl.ds((j % 8) * 16, 16)] = jnp.where(m, posb, posb + B)

  tables = (t0_hbm, t1_hbm, t2_hbm)
  nchunk = _NT * _NBUF

  def start_gather(k):
    t, q = k // _NBUF, k % _NBUF
    return pltpu.async_copy(
        tables[t].at[midx[t].at[pl.ds(q * _Q, _Q)]],
        rows_v.at[pl.ds((k % _NBUF) * _Q, _Q)], gsems[k % _NBUF])

  gh = [None] * nchunk
  wh = [None] * nchunk
  depth = _NBUF - 1
  for k in range(depth):
    gh[k] = start_gather(k)
  for k in range(nchunk):
    if k + depth < nchunk:
      if k + depth >= _NBUF:
        wh[k + depth - _NBUF].wait()
      gh[k + depth] = start_gather(k + depth)
    gh[k].wait()
    t, q = k // _NBUF, k % _NBUF
    wh[k] = pltpu.async_copy(
        rows_v.at[pl.ds((k % _NBUF) * _Q, _Q)],
        g_out.at[pos2[t].at[q]], wsems[k % _NBUF])
  for k in range(nchunk - _NBUF, nchunk):
    wh[k].wait()
  h_ty.wait()


@functools.cache
def _sc_gather():
  return pl.kernel(
      _sc_gather_body,
      out_type=[
          jax.ShapeDtypeStruct((B,), jnp.int32),
          jax.ShapeDtypeStruct((2 * B, FEAT), jnp.float32),
      ],
      mesh=plsc.VectorSubcoreMesh(core_axis_name="c", subcore_axis_name="s"),
      scratch_types=[
          pltpu.VMEM((BPW,), jnp.int32),
          pltpu.VMEM((BPW,), jnp.int32),
          pltpu.VMEM((BPW,), jnp.int32),
          pltpu.VMEM((BPW, FEAT), jnp.float32),
          pltpu.VMEM((BPW,), jnp.int32),
          pltpu.VMEM((BPW,), jnp.int32),
          pltpu.VMEM((BPW,), jnp.int32),
          pltpu.VMEM((_NBUF, _Q), jnp.int32),
          pltpu.VMEM((_NBUF, _Q), jnp.int32),
          pltpu.VMEM((_NBUF, _Q), jnp.int32),
      ] + [pltpu.SemaphoreType.DMA] * 11,
  )


_K2 = 2 * FEAT
_N2 = _NT * OUT_DIM
_BTC = 4096
_GRID = B // _BTC


def _tc_project_body(ty_ref, g_ref, w_ref, o_ref):
  ty = ty_ref[...].astype(jnp.int32)
  big = jnp.concatenate(
      [g_ref[...], jnp.ones((_BTC, FEAT), jnp.float32)], axis=1)
  p = jnp.dot(big, w_ref[...], preferred_element_type=jnp.float32)
  lanes = lax.broadcasted_iota(jnp.int32, (_BTC, OUT_DIM), 1)
  acc = (lanes == ty).astype(jnp.float32)
  for t in range(_NT):
    acc = acc + jnp.where(ty == t, p[:, t * OUT_DIM:(t + 1) * OUT_DIM], 0.0)
  o_ref[...] = acc


def _tc_project(ty_col, g, wc):
  return pl.pallas_call(
      _tc_project_body,
      grid=(_GRID,),
      in_specs=[
          pl.BlockSpec((_BTC, 1), lambda i: (i, 0)),
          pl.BlockSpec((_BTC, FEAT), lambda i: (i, 0)),
          pl.BlockSpec((_K2, _N2), lambda i: (0, 0)),
      ],
      out_specs=pl.BlockSpec((_BTC, OUT_DIM), lambda i: (i, 0)),
      out_shape=jax.ShapeDtypeStruct((B, OUT_DIM), jnp.float32),
  )(ty_col, g, wc)


def kernel(batch, node_types, feat_map, ip_feats, domain_feats, url_feats,
           W_ip, b_ip, W_dom, b_dom, W_url, b_url):
  batch = batch.astype(jnp.int32)
  node_types = node_types.astype(jnp.int32)
  feat_map = feat_map.astype(jnp.int32)

  ty_b, g = _sc_gather()(batch, node_types, feat_map,
                         ip_feats, domain_feats, url_feats)

  blocks = []
  for w, b in ((W_ip, b_ip), (W_dom, b_dom), (W_url, b_url)):
    top = jnp.pad(w.T.astype(jnp.float32), ((0, 0), (NUM_NODE_TYPES, 0)))
    brow = jnp.pad(b.astype(jnp.float32)[None, :],
                   ((0, 0), (NUM_NODE_TYPES, 0)))
    blocks.append(jnp.concatenate(
        [top, brow, jnp.zeros((FEAT - 1, OUT_DIM), jnp.float32)], axis=0))
  wc = jnp.concatenate(blocks, axis=1)

  ty_col = ty_b.astype(jnp.bfloat16).reshape(B, 1)
  return _tc_project(ty_col, g, wc)
